# baseline (speedup 1.0000x reference)
import jax
import jax.numpy as jnp
from jax import lax
from jax.experimental import pallas as pl
from jax.experimental.pallas import tpu as pltpu
from jax.experimental.pallas import tpu_sc as plsc

NUSERS = 100000
NITEMS = 100000
D = 16
B = 4096
LH = 50
LS = 20
NRAT = 5

SLH = 56
SLS = 24
NPH = SLH // 8
NPS = SLS // 8

NC = 2
NS = 16
NW = NC * NS
BPW = B // NW
SUB = 64
NSUB = BPW // SUB


def _sc_gather_body(nodes_u, nodes_p, nodes_n, hist_u, hist_ur, hist_v,
                    hist_vr, soc, u2e, v2e,
                    eh_u, eh_p, eh_n, rs_u, rs_p, rs_n, se_o,
                    rp_u, rp_p, rp_n,
                    nodes_v, hrows_v, rrows_v, srows_v, e_v, s_v,
                    rep_v, rsl_v, sem1, sem2, sem3):
  wid = lax.axis_index("s") * NC + lax.axis_index("c")
  zero = jnp.zeros((D,), jnp.float32)

  def zpad(i, carry):
    for j in range(LH, SLH):
      e_v[i * SLH + j, :] = zero
    for j in range(LS, SLS):
      s_v[i * SLS + j, :] = zero
    return carry

  lax.fori_loop(0, SUB, zpad, 0)

  def branch(base, nodes_hbm, hrow_hbm, rrow_hbm, rep_tab, emb_tab,
             eh_out, rs_out, rep_out, do_soc):
    pltpu.sync_copy(nodes_hbm.at[pl.ds(base, SUB)], nodes_v)
    d1 = pltpu.async_copy(hrow_hbm.at[nodes_v], hrows_v, sem1)
    d2 = pltpu.async_copy(rrow_hbm.at[nodes_v], rrows_v, sem2)
    d3 = pltpu.async_copy(rep_tab.at[nodes_v], rep_v, sem3)
    if do_soc:
      d4 = pltpu.async_copy(soc.at[nodes_v], srows_v, sem3)
    d1.wait()
    d2.wait()

    def step(g, carry):
      des = []
      for k in range(8):
        row = g * 8 + k
        des.append(pltpu.async_copy(emb_tab.at[hrows_v.at[row]],
                                    e_v.at[pl.ds(row * SLH, LH)], sem1))
        for off in (0, 16, 32, LH - 16):
          rsl_v[pl.ds(row * SLH + off, 16)] = rrows_v[row, pl.ds(off, 16)]
        if do_soc:
          des.append(pltpu.async_copy(u2e.at[srows_v.at[row]],
                                      s_v.at[pl.ds(row * SLS, LS)], sem3))
      for d in des:
        d.wait()
      return carry

    if do_soc:
      d4.wait()
    lax.fori_loop(0, SUB // 8, step, 0)
    d3.wait()
    pltpu.sync_copy(e_v, eh_out.at[pl.ds(base * SLH, SUB * SLH)])
    pltpu.sync_copy(rsl_v, rs_out.at[pl.ds(base * SLH, SUB * SLH)])
    pltpu.sync_copy(rep_v, rep_out.at[pl.ds(base, SUB)])
    if do_soc:
      pltpu.sync_copy(s_v, se_o.at[pl.ds(base * SLS, SUB * SLS)])

  for s in range(NSUB):
    base = wid * BPW + s * SUB
    branch(base, nodes_u, hist_u, hist_ur, u2e, v2e,
           eh_u, rs_u, rp_u, True)
    branch(base, nodes_p, hist_v, hist_vr, v2e, u2e,
           eh_p, rs_p, rp_p, False)
    branch(base, nodes_n, hist_v, hist_vr, v2e, u2e,
           eh_n, rs_n, rp_n, False)


def _sc_gather(nodes_u, nodes_p, nodes_n, hist_u, hist_ur, hist_v, hist_vr,
               soc, u2e, v2e):
  f32, i32 = jnp.float32, jnp.int32
  out_type = [
      jax.ShapeDtypeStruct((B * SLH, D), f32),
      jax.ShapeDtypeStruct((B * SLH, D), f32),
      jax.ShapeDtypeStruct((B * SLH, D), f32),
      jax.ShapeDtypeStruct((B * SLH,), i32),
      jax.ShapeDtypeStruct((B * SLH,), i32),
      jax.ShapeDtypeStruct((B * SLH,), i32),
      jax.ShapeDtypeStruct((B * SLS, D), f32),
      jax.ShapeDtypeStruct((B, D), f32),
      jax.ShapeDtypeStruct((B, D), f32),
      jax.ShapeDtypeStruct((B, D), f32),
  ]
  scratch = [
      pltpu.VMEM((SUB,), i32),
      pltpu.VMEM((SUB, LH), i32),
      pltpu.VMEM((SUB, LH), i32),
      pltpu.VMEM((SUB, LS), i32),
      pltpu.VMEM((SUB * SLH, D), f32),
      pltpu.VMEM((SUB * SLS, D), f32),
      pltpu.VMEM((SUB, D), f32),
      pltpu.VMEM((SUB * SLH,), i32),
      pltpu.SemaphoreType.DMA,
      pltpu.SemaphoreType.DMA,
      pltpu.SemaphoreType.DMA,
  ]
  fn = pl.kernel(
      _sc_gather_body,
      out_type=out_type,
      scratch_types=scratch,
      mesh=plsc.VectorSubcoreMesh(core_axis_name="c", subcore_axis_name="s"),
      compiler_params=pltpu.CompilerParams(use_tc_tiling_on_sc=False),
  )
  return fn(nodes_u, nodes_p, nodes_n, hist_u, hist_ur, hist_v, hist_vr,
            soc, u2e, v2e)


C = 512
G = B // C

_relu = lambda x: jnp.maximum(x, 0.0)


def _dot(a, b):
  return lax.dot_general(a, b, (((1,), (0,)), ((), ())),
                         preferred_element_type=jnp.float32)


def _att_softmax_fold(o, scores, mask, np_, cm):
  s3 = scores.reshape(C, np_, 8) + mask[None]
  s3 = s3 - jnp.max(jnp.max(s3, axis=2, keepdims=True), axis=1, keepdims=True)
  e3 = jnp.exp(s3)
  den = jnp.sum(jnp.sum(e3, axis=2, keepdims=True), axis=1, keepdims=True)
  att = e3 / den
  att_exp = _dot(att.reshape(C * np_, 8), cm["E16"])
  neigh_p = (o * att_exp).reshape(C, np_, 128).sum(axis=1)
  return _dot(neigh_p, cm["F"])


def _neigh_agg(ehp, rsl, rep, p, x, cm):
  rexp = _dot(rsl.astype(jnp.float32), cm["E5"])
  oh = (rexp == cm["K40"]).astype(jnp.float32)
  erp = _dot(oh, p[x + "T40"])
  h = _relu(_dot(ehp, p[x + "W1a"]) + erp + p[x + "b1"])
  o = _relu(_dot(h, p[x + "W2"]) + p[x + "b2"])
  rep_t = _dot(_dot(rep, p[x + "A1b"]) + p[x + "a1b"], cm["G16"])
  x1 = _relu((_dot(o, p[x + "A1a"]).reshape(C, NPH, 128) +
              rep_t[:, None, :]).reshape(C * NPH, 128))
  x2 = _relu(_dot(x1, p[x + "A2"]) + p[x + "a2b"])
  s = _dot(x2, p[x + "a3"])
  return _att_softmax_fold(o, s, p["maskh"], NPH, cm)


def _soc_agg(sp, rep, p, cm):
  rep_t = _dot(_dot(rep, p["S1b"]) + p["s1b"], cm["G16"])
  x1 = _relu((_dot(sp, p["S1a"]).reshape(C, NPS, 128) +
              rep_t[:, None, :]).reshape(C * NPS, 128))
  x2 = _relu(_dot(x1, p["S2"]) + p["s2b"])
  s = _dot(x2, p["s3"])
  return _att_softmax_fold(sp, s, p["masks"], NPS, cm)


def _tc1_body(ehp_u, rsl_u, rep_u, socp, ehp_p, rsl_p, rep_p,
              ehp_n, rsl_n, rep_n, pp, xu_o, xi_o, xj_o):
  p = jax.tree.map(lambda r: r[...], pp)
  rep_u_, rep_p_, rep_n_ = rep_u[...], rep_p[...], rep_n[...]

  nu = _neigh_agg(ehp_u[...], rsl_u[...], rep_u_, p, "u_", p)
  self_u = _relu(_dot(rep_u_, p["EuhA"]) + _dot(nu, p["EuhB"]) + p["euhb"])
  ns = _soc_agg(socp[...], rep_u_, p, p)
  emb_u = _relu(_dot(self_u, p["EuA"]) + _dot(ns, p["EuB"]) + p["eub"])
  xu_o[...] = _dot(emb_u, p["Wur1"]) + p["bur1"]

  np_ = _neigh_agg(ehp_p[...], rsl_p[...], rep_p_, p, "v_", p)
  emb_i = _relu(_dot(rep_p_, p["EvhA"]) + _dot(np_, p["EvhB"]) + p["evhb"])
  xi_o[...] = _dot(emb_i, p["Wvr1"]) + p["bvr1"]

  nn = _neigh_agg(ehp_n[...], rsl_n[...], rep_n_, p, "v_", p)
  emb_j = _relu(_dot(rep_n_, p["EvhA"]) + _dot(nn, p["EvhB"]) + p["evhb"])
  xj_o[...] = _dot(emb_j, p["Wvr1"]) + p["bvr1"]


def _tc2_body(xu, xi, xj, pp, out):
  p = jax.tree.map(lambda r: r[...], pp)

  def bn_head(x, g, b, w, bo):
    mean = jnp.mean(x, axis=0, keepdims=True)
    var = jnp.mean((x - mean) ** 2, axis=0, keepdims=True)
    xn = g * (x - mean) / jnp.sqrt(var + 1e-5) + b
    return _dot(_relu(xn), w) + bo

  x_u = bn_head(xu[...], p["g1"], p["b1"], p["Wur2"], p["bur2"])
  x_i = bn_head(xi[...], p["g2"], p["b2"], p["Wvr2"], p["bvr2"])
  x_j = bn_head(xj[...], p["g2"], p["b2"], p["Wvr2"], p["bvr2"])
  d = jnp.sum(x_u * x_i - x_u * x_j, axis=1)
  lp = jnp.sum(jnp.minimum(d, 0.0) - jnp.log(1.0 + jnp.exp(-jnp.abs(d))))
  reg = 1e-4 * (jnp.sum(x_u ** 2) + jnp.sum(x_i ** 2) + jnp.sum(x_j ** 2))
  out[...] = jnp.reshape(reg - lp, (1, 1))


def _prep_params(P):
  f32 = jnp.float32
  I8 = jnp.eye(8, dtype=f32)
  bd = lambda w: jnp.kron(I8, w)
  tile = lambda v: jnp.tile(v, 8)[None]

  def split2(w):
    return w[:D], w[D:]

  pr = {}
  tabs = {}
  for tag, agg in (("u_", P["agg_u"]), ("v_", P["agg_v"])):
    w1a, w1b = split2(agg["w_r1_w"])
    tabs[tag] = P["r2e"] @ w1b
    pr[tag + "T40"] = bd(tabs[tag])
    att = agg["att"]
    a1a, a1b_w = split2(att["a1w"])
    pr[tag + "W1a"] = bd(w1a)
    pr[tag + "b1"] = tile(agg["w_r1_b"])
    pr[tag + "W2"] = bd(agg["w_r2_w"])
    pr[tag + "b2"] = tile(agg["w_r2_b"])
    pr[tag + "A1a"] = bd(a1a)
    pr[tag + "A1b"] = a1b_w
    pr[tag + "a1b"] = att["a1b"][None]
    pr[tag + "A2"] = bd(att["a2w"])
    pr[tag + "a2b"] = tile(att["a2b"])
    pr[tag + "a3"] = bd(att["a3w"])
  s1a, s1b_w = split2(P["soc_att"]["a1w"])
  pr["S1a"], pr["S1b"] = bd(s1a), s1b_w
  pr["s1b"] = P["soc_att"]["a1b"][None]
  pr["S2"], pr["s2b"] = bd(P["soc_att"]["a2w"]), tile(P["soc_att"]["a2b"])
  pr["s3"] = bd(P["soc_att"]["a3w"])
  for nm, key in (("Euh", "enc_uh"), ("Evh", "enc_vh"), ("Eu", "enc_u")):
    wa, wb = split2(P[key + "_w"])
    pr[nm + "A"], pr[nm + "B"] = wa, wb
  pr["euhb"] = P["enc_uh_b"][None]
  pr["evhb"] = P["enc_vh_b"][None]
  pr["eub"] = P["enc_u_b"][None]
  pr["Wur1"], pr["bur1"] = P["w_ur1_w"], P["w_ur1_b"][None]
  pr["Wvr1"], pr["bvr1"] = P["w_vr1_w"], P["w_vr1_b"][None]
  pr["E16"] = jnp.kron(I8, jnp.ones((1, D), f32))
  pr["E5"] = jnp.kron(I8, jnp.ones((1, NRAT), f32))
  pr["K40"] = (jnp.arange(8 * NRAT) % NRAT).astype(f32)[None]
  pr["F"] = jnp.kron(jnp.ones((8, 1), f32), jnp.eye(D))
  pr["G16"] = jnp.kron(jnp.ones((1, 8), f32), jnp.eye(D))
  nidx = jnp.arange(SLH).reshape(NPH, 8)
  pr["maskh"] = jnp.where(nidx < LH, 0.0, -1e30).astype(f32)
  sidx = jnp.arange(SLS).reshape(NPS, 8)
  pr["masks"] = jnp.where(sidx < LS, 0.0, -1e30).astype(f32)
  p2 = dict(Wur2=P["w_ur2_w"], bur2=P["w_ur2_b"][None],
            Wvr2=P["w_vr2_w"], bvr2=P["w_vr2_b"][None],
            g1=P["bn1_g"][None], b1=P["bn1_b"][None],
            g2=P["bn2_g"][None], b2=P["bn2_b"][None])
  return pr, p2, tabs["u_"], tabs["v_"]


def _full(x):
  return pl.BlockSpec(x.shape, lambda *_: (0,) * x.ndim)


def _tc_stage1(ehp_u, rsl_u, rep_u, socp, ehp_p, rsl_p, rep_p,
               ehp_n, rsl_n, rep_n, pr):
  eh_spec = pl.BlockSpec((C * NPH, 128), lambda i: (i, 0))
  rsl_spec = pl.BlockSpec((C * NPH, 8), lambda i: (i, 0))
  rep_spec = pl.BlockSpec((C, D), lambda i: (i, 0))
  soc_spec = pl.BlockSpec((C * NPS, 128), lambda i: (i, 0))
  f32 = jnp.float32
  return pl.pallas_call(
      _tc1_body,
      grid=(G,),
      in_specs=[eh_spec, rsl_spec, rep_spec, soc_spec,
                eh_spec, rsl_spec, rep_spec,
                eh_spec, rsl_spec, rep_spec, jax.tree.map(_full, pr)],
      out_specs=[rep_spec, rep_spec, rep_spec],
      out_shape=[jax.ShapeDtypeStruct((B, D), f32)] * 3,
  )(ehp_u, rsl_u, rep_u, socp, ehp_p, rsl_p, rep_p,
    ehp_n, rsl_n, rep_n, pr)


def _tc_stage2(xu, xi, xj, p2):
  x_spec = pl.BlockSpec((B, D), lambda: (0, 0))
  return pl.pallas_call(
      _tc2_body,
      in_specs=[x_spec, x_spec, x_spec, jax.tree.map(_full, p2)],
      out_specs=pl.BlockSpec((1, 1), lambda: (0, 0)),
      out_shape=jax.ShapeDtypeStruct((1, 1), jnp.float32),
  )(xu, xi, xj, p2)


def kernel(nodes_u, nodes_pos, nodes_neg, hist_u, hist_ur, hist_v, hist_vr,
           soc_adj, params):
  pr, p2, _, _ = _prep_params(params)
  (eh_u, eh_p, eh_n, rs_u, rs_p, rs_n, soce, rp_u, rp_p, rp_n) = _sc_gather(
      nodes_u, nodes_pos, nodes_neg, hist_u, hist_ur, hist_v, hist_vr,
      soc_adj, params["u2e"], params["v2e"])
  fl = lambda a: a.reshape(-1, 128)
  r8 = lambda a: a.reshape(-1, 8)
  xu, xi, xj = _tc_stage1(fl(eh_u), r8(rs_u), rp_u, fl(soce),
                          fl(eh_p), r8(rs_p), rp_p,
                          fl(eh_n), r8(rs_n), rp_n, pr)
  return _tc_stage2(xu, xi, xj, p2)[0, 0]


# speedup vs baseline: 1.4100x; 1.4100x over previous
import jax
import jax.numpy as jnp
from jax import lax
from jax.experimental import pallas as pl
from jax.experimental.pallas import tpu as pltpu
from jax.experimental.pallas import tpu_sc as plsc

NUSERS = 100000
NITEMS = 100000
D = 16
B = 4096
LH = 50
LS = 20
NRAT = 5

SLH = 56
SLS = 24
NPH = SLH // 8
NPS = SLS // 8

NC = 2
NS = 16
NW = NC * NS
BPW = B // NW
SUB = 64
NSUB = BPW // SUB


def _make_sc_body(nb):
  bpw = nb // NW
  nsub = max(bpw // SUB, 1)
  sub = min(SUB, bpw)

  def _sc_gather_body(nodes_u, nodes_p, nodes_n, hist_u, hist_ur, hist_v,
                      hist_vr, soc, u2e, v2e,
                      eh_u, eh_p, eh_n, rs_u, rs_p, rs_n, se_o,
                      rp_u, rp_p, rp_n,
                      nodes_v, hrows_v, rrows_v, srows_v, e_v, s_v,
                      rep_v, rsl_v, sem1, sem2, sem3):
    wid = lax.axis_index("s") * NC + lax.axis_index("c")
    zero = jnp.zeros((D,), jnp.float32)

    def zpad(i, carry):
      for j in range(LH, SLH):
        e_v[i * SLH + j, :] = zero
      for j in range(LS, SLS):
        s_v[i * SLS + j, :] = zero
      return carry

    lax.fori_loop(0, sub, zpad, 0)

    def branch(base, nodes_hbm, hrow_hbm, rrow_hbm, rep_tab, emb_tab,
               eh_out, rs_out, rep_out, do_soc):
      pltpu.sync_copy(nodes_hbm.at[pl.ds(base, sub)], nodes_v)
      d1 = pltpu.async_copy(hrow_hbm.at[nodes_v], hrows_v, sem1)
      d2 = pltpu.async_copy(rrow_hbm.at[nodes_v], rrows_v, sem2)
      d3 = pltpu.async_copy(rep_tab.at[nodes_v], rep_v, sem3)
      if do_soc:
        d4 = pltpu.async_copy(soc.at[nodes_v], srows_v, sem3)
      d1.wait()
      d2.wait()

      def step(g, carry):
        des = []
        for k in range(8):
          row = g * 8 + k
          des.append(pltpu.async_copy(emb_tab.at[hrows_v.at[row]],
                                      e_v.at[pl.ds(row * SLH, LH)], sem1))
          for off in (0, 16, 32, LH - 16):
            rsl_v[pl.ds(row * SLH + off, 16)] = rrows_v[row, pl.ds(off, 16)]
          if do_soc:
            des.append(pltpu.async_copy(u2e.at[srows_v.at[row]],
                                        s_v.at[pl.ds(row * SLS, LS)], sem3))
        for d in des:
          d.wait()
        return carry

      if do_soc:
        d4.wait()
      lax.fori_loop(0, sub // 8, step, 0)
      d3.wait()
      pltpu.sync_copy(e_v, eh_out.at[pl.ds(base * SLH, sub * SLH)])
      pltpu.sync_copy(rsl_v, rs_out.at[pl.ds(base * SLH, sub * SLH)])
      pltpu.sync_copy(rep_v, rep_out.at[pl.ds(base, sub)])
      if do_soc:
        pltpu.sync_copy(s_v, se_o.at[pl.ds(base * SLS, sub * SLS)])

    for s in range(nsub):
      base = wid * bpw + s * sub
      branch(base, nodes_u, hist_u, hist_ur, u2e, v2e,
             eh_u, rs_u, rp_u, True)
      branch(base, nodes_p, hist_v, hist_vr, v2e, u2e,
             eh_p, rs_p, rp_p, False)
      branch(base, nodes_n, hist_v, hist_vr, v2e, u2e,
             eh_n, rs_n, rp_n, False)

  return _sc_gather_body, sub


def _sc_gather(nodes_u, nodes_p, nodes_n, hist_u, hist_ur, hist_v, hist_vr,
               soc, u2e, v2e):
  f32, i32 = jnp.float32, jnp.int32
  nb = nodes_u.shape[0]
  body, sub = _make_sc_body(nb)
  out_type = [
      jax.ShapeDtypeStruct((nb * SLH, D), f32),
      jax.ShapeDtypeStruct((nb * SLH, D), f32),
      jax.ShapeDtypeStruct((nb * SLH, D), f32),
      jax.ShapeDtypeStruct((nb * SLH,), i32),
      jax.ShapeDtypeStruct((nb * SLH,), i32),
      jax.ShapeDtypeStruct((nb * SLH,), i32),
      jax.ShapeDtypeStruct((nb * SLS, D), f32),
      jax.ShapeDtypeStruct((nb, D), f32),
      jax.ShapeDtypeStruct((nb, D), f32),
      jax.ShapeDtypeStruct((nb, D), f32),
  ]
  scratch = [
      pltpu.VMEM((sub,), i32),
      pltpu.VMEM((sub, LH), i32),
      pltpu.VMEM((sub, LH), i32),
      pltpu.VMEM((sub, LS), i32),
      pltpu.VMEM((sub * SLH, D), f32),
      pltpu.VMEM((sub * SLS, D), f32),
      pltpu.VMEM((sub, D), f32),
      pltpu.VMEM((sub * SLH,), i32),
      pltpu.SemaphoreType.DMA,
      pltpu.SemaphoreType.DMA,
      pltpu.SemaphoreType.DMA,
  ]
  fn = pl.kernel(
      body,
      out_type=out_type,
      scratch_types=scratch,
      mesh=plsc.VectorSubcoreMesh(core_axis_name="c", subcore_axis_name="s"),
      compiler_params=pltpu.CompilerParams(use_tc_tiling_on_sc=False),
  )
  return fn(nodes_u, nodes_p, nodes_n, hist_u, hist_ur, hist_v, hist_vr,
            soc, u2e, v2e)


C = 256
G = B // C

_relu = lambda x: jnp.maximum(x, 0.0)


def _dot(a, b):
  return lax.dot_general(a, b, (((1,), (0,)), ((), ())),
                         preferred_element_type=jnp.float32)


def _att_softmax_fold(o, scores, mask, np_, cm):
  s3 = scores.reshape(C, np_, 8) + mask[None]
  s3 = s3 - jnp.max(jnp.max(s3, axis=2, keepdims=True), axis=1, keepdims=True)
  e3 = jnp.exp(s3)
  den = jnp.sum(jnp.sum(e3, axis=2, keepdims=True), axis=1, keepdims=True)
  att = e3 / den
  att_exp = _dot(att.reshape(C * np_, 8), cm["E16"])
  neigh_p = (o * att_exp).reshape(C, np_, 128).sum(axis=1)
  return _dot(neigh_p, cm["F"])


def _neigh_agg(ehp, rsl, rep, p, x, cm):
  rexp = _dot(rsl.astype(jnp.float32), cm["E5"])
  oh = (rexp == cm["K40"]).astype(jnp.float32)
  erp = _dot(oh, p[x + "T40"])
  h = _relu(_dot(ehp, p[x + "W1a"]) + erp + p[x + "b1"])
  o = _relu(_dot(h, p[x + "W2"]) + p[x + "b2"])
  rep_t = _dot(_dot(rep, p[x + "A1b"]) + p[x + "a1b"], cm["G16"])
  x1 = _relu((_dot(o, p[x + "A1a"]).reshape(C, NPH, 128) +
              rep_t[:, None, :]).reshape(C * NPH, 128))
  x2 = _relu(_dot(x1, p[x + "A2"]) + p[x + "a2b"])
  s = _dot(x2, p[x + "a3"])
  return _att_softmax_fold(o, s, p["maskh"], NPH, cm)


def _soc_agg(sp, rep, p, cm):
  rep_t = _dot(_dot(rep, p["S1b"]) + p["s1b"], cm["G16"])
  x1 = _relu((_dot(sp, p["S1a"]).reshape(C, NPS, 128) +
              rep_t[:, None, :]).reshape(C * NPS, 128))
  x2 = _relu(_dot(x1, p["S2"]) + p["s2b"])
  s = _dot(x2, p["s3"])
  return _att_softmax_fold(sp, s, p["masks"], NPS, cm)


def _tc1_body(ehp_u, rsl_u, rep_u, socp, ehp_p, rsl_p, rep_p,
              ehp_n, rsl_n, rep_n, pp, xu_o, xi_o, xj_o):
  p = jax.tree.map(lambda r: r[...], pp)
  rep_u_, rep_p_, rep_n_ = rep_u[...], rep_p[...], rep_n[...]

  nu = _neigh_agg(ehp_u[...], rsl_u[...], rep_u_, p, "u_", p)
  self_u = _relu(_dot(rep_u_, p["EuhA"]) + _dot(nu, p["EuhB"]) + p["euhb"])
  ns = _soc_agg(socp[...], rep_u_, p, p)
  emb_u = _relu(_dot(self_u, p["EuA"]) + _dot(ns, p["EuB"]) + p["eub"])
  xu_o[...] = _dot(emb_u, p["Wur1"]) + p["bur1"]

  np_ = _neigh_agg(ehp_p[...], rsl_p[...], rep_p_, p, "v_", p)
  emb_i = _relu(_dot(rep_p_, p["EvhA"]) + _dot(np_, p["EvhB"]) + p["evhb"])
  xi_o[...] = _dot(emb_i, p["Wvr1"]) + p["bvr1"]

  nn = _neigh_agg(ehp_n[...], rsl_n[...], rep_n_, p, "v_", p)
  emb_j = _relu(_dot(rep_n_, p["EvhA"]) + _dot(nn, p["EvhB"]) + p["evhb"])
  xj_o[...] = _dot(emb_j, p["Wvr1"]) + p["bvr1"]


def _tc2_body(xu, xi, xj, pp, out):
  p = jax.tree.map(lambda r: r[...], pp)

  def bn_head(x, g, b, w, bo):
    mean = jnp.mean(x, axis=0, keepdims=True)
    var = jnp.mean((x - mean) ** 2, axis=0, keepdims=True)
    xn = g * (x - mean) / jnp.sqrt(var + 1e-5) + b
    return _dot(_relu(xn), w) + bo

  x_u = bn_head(xu[...], p["g1"], p["b1"], p["Wur2"], p["bur2"])
  x_i = bn_head(xi[...], p["g2"], p["b2"], p["Wvr2"], p["bvr2"])
  x_j = bn_head(xj[...], p["g2"], p["b2"], p["Wvr2"], p["bvr2"])
  d = jnp.sum(x_u * x_i - x_u * x_j, axis=1)
  lp = jnp.sum(jnp.minimum(d, 0.0) - jnp.log(1.0 + jnp.exp(-jnp.abs(d))))
  reg = 1e-4 * (jnp.sum(x_u ** 2) + jnp.sum(x_i ** 2) + jnp.sum(x_j ** 2))
  out[...] = jnp.reshape(reg - lp, (1, 1))


def _prep_params(P):
  f32 = jnp.float32
  I8 = jnp.eye(8, dtype=f32)
  bd = lambda w: jnp.kron(I8, w)
  tile = lambda v: jnp.tile(v, 8)[None]

  def split2(w):
    return w[:D], w[D:]

  pr = {}
  tabs = {}
  for tag, agg in (("u_", P["agg_u"]), ("v_", P["agg_v"])):
    w1a, w1b = split2(agg["w_r1_w"])
    tabs[tag] = P["r2e"] @ w1b
    pr[tag + "T40"] = bd(tabs[tag])
    att = agg["att"]
    a1a, a1b_w = split2(att["a1w"])
    pr[tag + "W1a"] = bd(w1a)
    pr[tag + "b1"] = tile(agg["w_r1_b"])
    pr[tag + "W2"] = bd(agg["w_r2_w"])
    pr[tag + "b2"] = tile(agg["w_r2_b"])
    pr[tag + "A1a"] = bd(a1a)
    pr[tag + "A1b"] = a1b_w
    pr[tag + "a1b"] = att["a1b"][None]
    pr[tag + "A2"] = bd(att["a2w"])
    pr[tag + "a2b"] = tile(att["a2b"])
    pr[tag + "a3"] = bd(att["a3w"])
  s1a, s1b_w = split2(P["soc_att"]["a1w"])
  pr["S1a"], pr["S1b"] = bd(s1a), s1b_w
  pr["s1b"] = P["soc_att"]["a1b"][None]
  pr["S2"], pr["s2b"] = bd(P["soc_att"]["a2w"]), tile(P["soc_att"]["a2b"])
  pr["s3"] = bd(P["soc_att"]["a3w"])
  for nm, key in (("Euh", "enc_uh"), ("Evh", "enc_vh"), ("Eu", "enc_u")):
    wa, wb = split2(P[key + "_w"])
    pr[nm + "A"], pr[nm + "B"] = wa, wb
  pr["euhb"] = P["enc_uh_b"][None]
  pr["evhb"] = P["enc_vh_b"][None]
  pr["eub"] = P["enc_u_b"][None]
  pr["Wur1"], pr["bur1"] = P["w_ur1_w"], P["w_ur1_b"][None]
  pr["Wvr1"], pr["bvr1"] = P["w_vr1_w"], P["w_vr1_b"][None]
  pr["E16"] = jnp.kron(I8, jnp.ones((1, D), f32))
  pr["E5"] = jnp.kron(I8, jnp.ones((1, NRAT), f32))
  pr["K40"] = (jnp.arange(8 * NRAT) % NRAT).astype(f32)[None]
  pr["F"] = jnp.kron(jnp.ones((8, 1), f32), jnp.eye(D))
  pr["G16"] = jnp.kron(jnp.ones((1, 8), f32), jnp.eye(D))
  nidx = jnp.arange(SLH).reshape(NPH, 8)
  pr["maskh"] = jnp.where(nidx < LH, 0.0, -1e30).astype(f32)
  sidx = jnp.arange(SLS).reshape(NPS, 8)
  pr["masks"] = jnp.where(sidx < LS, 0.0, -1e30).astype(f32)
  p2 = dict(Wur2=P["w_ur2_w"], bur2=P["w_ur2_b"][None],
            Wvr2=P["w_vr2_w"], bvr2=P["w_vr2_b"][None],
            g1=P["bn1_g"][None], b1=P["bn1_b"][None],
            g2=P["bn2_g"][None], b2=P["bn2_b"][None])
  return pr, p2, tabs["u_"], tabs["v_"]


def _full(x):
  return pl.BlockSpec(x.shape, lambda *_: (0,) * x.ndim)


def _tc_stage1(ehp_u, rsl_u, rep_u, socp, ehp_p, rsl_p, rep_p,
               ehp_n, rsl_n, rep_n, pr):
  eh_spec = pl.BlockSpec((C * NPH, 128), lambda i: (i, 0))
  rsl_spec = pl.BlockSpec((C * NPH, 8), lambda i: (i, 0))
  rep_spec = pl.BlockSpec((C, D), lambda i: (i, 0))
  soc_spec = pl.BlockSpec((C * NPS, 128), lambda i: (i, 0))
  f32 = jnp.float32
  nb = rep_u.shape[0]
  return pl.pallas_call(
      _tc1_body,
      grid=(nb // C,),
      in_specs=[eh_spec, rsl_spec, rep_spec, soc_spec,
                eh_spec, rsl_spec, rep_spec,
                eh_spec, rsl_spec, rep_spec, jax.tree.map(_full, pr)],
      out_specs=[rep_spec, rep_spec, rep_spec],
      out_shape=[jax.ShapeDtypeStruct((nb, D), f32)] * 3,
  )(ehp_u, rsl_u, rep_u, socp, ehp_p, rsl_p, rep_p,
    ehp_n, rsl_n, rep_n, pr)


def _tc_stage2(xu, xi, xj, p2):
  x_spec = pl.BlockSpec((B, D), lambda: (0, 0))
  return pl.pallas_call(
      _tc2_body,
      in_specs=[x_spec, x_spec, x_spec, jax.tree.map(_full, p2)],
      out_specs=pl.BlockSpec((1, 1), lambda: (0, 0)),
      out_shape=jax.ShapeDtypeStruct((1, 1), jnp.float32),
  )(xu, xi, xj, p2)


def kernel(nodes_u, nodes_pos, nodes_neg, hist_u, hist_ur, hist_v, hist_vr,
           soc_adj, params):
  pr, p2, _, _ = _prep_params(params)
  fl = lambda a: a.reshape(-1, 128)
  r8 = lambda a: a.reshape(-1, 8)
  h = B // 2
  xs = []
  for lo in (0, h):
    (eh_u, eh_p, eh_n, rs_u, rs_p, rs_n, soce, rp_u, rp_p,
     rp_n) = _sc_gather(
        lax.dynamic_slice_in_dim(nodes_u, lo, h),
        lax.dynamic_slice_in_dim(nodes_pos, lo, h),
        lax.dynamic_slice_in_dim(nodes_neg, lo, h),
        hist_u, hist_ur, hist_v, hist_vr, soc_adj,
        params["u2e"], params["v2e"])
    xs.append(_tc_stage1(fl(eh_u), r8(rs_u), rp_u, fl(soce),
                         fl(eh_p), r8(rs_p), rp_p,
                         fl(eh_n), r8(rs_n), rp_n, pr))
  xu, xi, xj = (jnp.concatenate([a, b]) for a, b in zip(*xs))
  return _tc_stage2(xu, xi, xj, p2)[0, 0]


# speedup vs baseline: 1.4717x; 1.0438x over previous
import jax
import jax.numpy as jnp
from jax import lax
from jax.experimental import pallas as pl
from jax.experimental.pallas import tpu as pltpu
from jax.experimental.pallas import tpu_sc as plsc

NUSERS = 100000
NITEMS = 100000
D = 16
B = 4096
LH = 50
LS = 20
NRAT = 5

SLH = 56
SLS = 24
NPH = SLH // 8
NPS = SLS // 8

NC = 2
NS = 16
NW = NC * NS
BPW = B // NW
SUB = 64
NSUB = BPW // SUB


def _make_sc_body(nb):
  bpw = nb // NW
  nsub = max(bpw // SUB, 1)
  sub = min(SUB, bpw)

  def _sc_gather_body(nodes_u, nodes_p, nodes_n, hist_u, hist_ur, hist_v,
                      hist_vr, soc, u2e, v2e,
                      eh_u, eh_p, eh_n, rs_u, rs_p, rs_n, se_o,
                      rp_u, rp_p, rp_n,
                      nodes_v, hrows_v, rrows_v, srows_v, e_v, s_v,
                      rep_v, rsl_v, sem1, sem2, sem3):
    wid = lax.axis_index("s") * NC + lax.axis_index("c")
    zero = jnp.zeros((D,), jnp.float32)

    def zpad(i, carry):
      for j in range(LH, SLH):
        e_v[i * SLH + j, :] = zero
      for j in range(LS, SLS):
        s_v[i * SLS + j, :] = zero
      return carry

    lax.fori_loop(0, sub, zpad, 0)

    def branch(base, nodes_hbm, hrow_hbm, rrow_hbm, rep_tab, emb_tab,
               eh_out, rs_out, rep_out, do_soc):
      pltpu.sync_copy(nodes_hbm.at[pl.ds(base, sub)], nodes_v)
      d1 = pltpu.async_copy(hrow_hbm.at[nodes_v], hrows_v, sem1)
      d2 = pltpu.async_copy(rrow_hbm.at[nodes_v], rrows_v, sem2)
      d3 = pltpu.async_copy(rep_tab.at[nodes_v], rep_v, sem3)
      if do_soc:
        d4 = pltpu.async_copy(soc.at[nodes_v], srows_v, sem3)
      d1.wait()
      d2.wait()

      def step(g, carry):
        des = []
        for k in range(8):
          row = g * 8 + k
          des.append(pltpu.async_copy(emb_tab.at[hrows_v.at[row]],
                                      e_v.at[pl.ds(row * SLH, LH)], sem1))
          for off in (0, 16, 32, LH - 16):
            rsl_v[pl.ds(row * SLH + off, 16)] = rrows_v[row, pl.ds(off, 16)]
          if do_soc:
            des.append(pltpu.async_copy(u2e.at[srows_v.at[row]],
                                        s_v.at[pl.ds(row * SLS, LS)], sem3))
        for d in des:
          d.wait()
        return carry

      if do_soc:
        d4.wait()
      lax.fori_loop(0, sub // 8, step, 0)
      d3.wait()
      pltpu.sync_copy(e_v, eh_out.at[pl.ds(base * SLH, sub * SLH)])
      pltpu.sync_copy(rsl_v, rs_out.at[pl.ds(base * SLH, sub * SLH)])
      pltpu.sync_copy(rep_v, rep_out.at[pl.ds(base, sub)])
      if do_soc:
        pltpu.sync_copy(s_v, se_o.at[pl.ds(base * SLS, sub * SLS)])

    for s in range(nsub):
      base = wid * bpw + s * sub
      branch(base, nodes_u, hist_u, hist_ur, u2e, v2e,
             eh_u, rs_u, rp_u, True)
      branch(base, nodes_p, hist_v, hist_vr, v2e, u2e,
             eh_p, rs_p, rp_p, False)
      branch(base, nodes_n, hist_v, hist_vr, v2e, u2e,
             eh_n, rs_n, rp_n, False)

  return _sc_gather_body, sub


def _sc_gather(nodes_u, nodes_p, nodes_n, hist_u, hist_ur, hist_v, hist_vr,
               soc, u2e, v2e):
  f32, i32 = jnp.float32, jnp.int32
  nb = nodes_u.shape[0]
  body, sub = _make_sc_body(nb)
  out_type = [
      jax.ShapeDtypeStruct((nb * SLH, D), f32),
      jax.ShapeDtypeStruct((nb * SLH, D), f32),
      jax.ShapeDtypeStruct((nb * SLH, D), f32),
      jax.ShapeDtypeStruct((nb * SLH,), i32),
      jax.ShapeDtypeStruct((nb * SLH,), i32),
      jax.ShapeDtypeStruct((nb * SLH,), i32),
      jax.ShapeDtypeStruct((nb * SLS, D), f32),
      jax.ShapeDtypeStruct((nb, D), f32),
      jax.ShapeDtypeStruct((nb, D), f32),
      jax.ShapeDtypeStruct((nb, D), f32),
  ]
  scratch = [
      pltpu.VMEM((sub,), i32),
      pltpu.VMEM((sub, LH), i32),
      pltpu.VMEM((sub, LH), i32),
      pltpu.VMEM((sub, LS), i32),
      pltpu.VMEM((sub * SLH, D), f32),
      pltpu.VMEM((sub * SLS, D), f32),
      pltpu.VMEM((sub, D), f32),
      pltpu.VMEM((sub * SLH,), i32),
      pltpu.SemaphoreType.DMA,
      pltpu.SemaphoreType.DMA,
      pltpu.SemaphoreType.DMA,
  ]
  fn = pl.kernel(
      body,
      out_type=out_type,
      scratch_types=scratch,
      mesh=plsc.VectorSubcoreMesh(core_axis_name="c", subcore_axis_name="s"),
      compiler_params=pltpu.CompilerParams(use_tc_tiling_on_sc=False),
  )
  return fn(nodes_u, nodes_p, nodes_n, hist_u, hist_ur, hist_v, hist_vr,
            soc, u2e, v2e)


C = 256
G = B // C

_relu = lambda x: jnp.maximum(x, 0.0)


def _dot(a, b):
  return lax.dot_general(a, b, (((1,), (0,)), ((), ())),
                         preferred_element_type=jnp.float32)


def _att_softmax_fold(o, scores, mask, np_, cm):
  s3 = scores.reshape(C, np_, 8) + mask[None]
  s3 = s3 - jnp.max(jnp.max(s3, axis=2, keepdims=True), axis=1, keepdims=True)
  e3 = jnp.exp(s3)
  den = jnp.sum(jnp.sum(e3, axis=2, keepdims=True), axis=1, keepdims=True)
  att = e3 / den
  att_exp = _dot(att.reshape(C * np_, 8), cm["E16"])
  neigh_p = (o * att_exp).reshape(C, np_, 128).sum(axis=1)
  return _dot(neigh_p, cm["F"])


def _neigh_agg(ehp, rsl, rep, p, x, cm):
  rexp = _dot(rsl.astype(jnp.float32), cm["E5"])
  oh = (rexp == cm["K40"]).astype(jnp.float32)
  erp = _dot(oh, p[x + "T40"])
  h = _relu(_dot(ehp, p[x + "W1a"]) + erp + p[x + "b1"])
  o = _relu(_dot(h, p[x + "W2"]) + p[x + "b2"])
  rep_t = _dot(_dot(rep, p[x + "A1b"]) + p[x + "a1b"], cm["G16"])
  x1 = _relu((_dot(o, p[x + "A1a"]).reshape(C, NPH, 128) +
              rep_t[:, None, :]).reshape(C * NPH, 128))
  x2 = _relu(_dot(x1, p[x + "A2"]) + p[x + "a2b"])
  s = _dot(x2, p[x + "a3"])
  return _att_softmax_fold(o, s, p["maskh"], NPH, cm)


def _soc_agg(sp, rep, p, cm):
  rep_t = _dot(_dot(rep, p["S1b"]) + p["s1b"], cm["G16"])
  x1 = _relu((_dot(sp, p["S1a"]).reshape(C, NPS, 128) +
              rep_t[:, None, :]).reshape(C * NPS, 128))
  x2 = _relu(_dot(x1, p["S2"]) + p["s2b"])
  s = _dot(x2, p["s3"])
  return _att_softmax_fold(sp, s, p["masks"], NPS, cm)


def _tc1_body(ehp_u, rsl_u, rep_u, socp, ehp_p, rsl_p, rep_p,
              ehp_n, rsl_n, rep_n, pp, xu_o, xi_o, xj_o):
  p = jax.tree.map(lambda r: r[...], pp)
  rep_u_, rep_p_, rep_n_ = rep_u[...], rep_p[...], rep_n[...]

  nu = _neigh_agg(ehp_u[...], rsl_u[...], rep_u_, p, "u_", p)
  self_u = _relu(_dot(rep_u_, p["EuhA"]) + _dot(nu, p["EuhB"]) + p["euhb"])
  ns = _soc_agg(socp[...], rep_u_, p, p)
  emb_u = _relu(_dot(self_u, p["EuA"]) + _dot(ns, p["EuB"]) + p["eub"])
  xu_o[...] = _dot(emb_u, p["Wur1"]) + p["bur1"]

  np_ = _neigh_agg(ehp_p[...], rsl_p[...], rep_p_, p, "v_", p)
  emb_i = _relu(_dot(rep_p_, p["EvhA"]) + _dot(np_, p["EvhB"]) + p["evhb"])
  xi_o[...] = _dot(emb_i, p["Wvr1"]) + p["bvr1"]

  nn = _neigh_agg(ehp_n[...], rsl_n[...], rep_n_, p, "v_", p)
  emb_j = _relu(_dot(rep_n_, p["EvhA"]) + _dot(nn, p["EvhB"]) + p["evhb"])
  xj_o[...] = _dot(emb_j, p["Wvr1"]) + p["bvr1"]


def _tc2_body(xu, xi, xj, pp, out):
  p = jax.tree.map(lambda r: r[...], pp)

  def bn_head(x, g, b, w, bo):
    mean = jnp.mean(x, axis=0, keepdims=True)
    var = jnp.mean((x - mean) ** 2, axis=0, keepdims=True)
    xn = g * (x - mean) / jnp.sqrt(var + 1e-5) + b
    return _dot(_relu(xn), w) + bo

  x_u = bn_head(xu[...], p["g1"], p["b1"], p["Wur2"], p["bur2"])
  x_i = bn_head(xi[...], p["g2"], p["b2"], p["Wvr2"], p["bvr2"])
  x_j = bn_head(xj[...], p["g2"], p["b2"], p["Wvr2"], p["bvr2"])
  d = jnp.sum(x_u * x_i - x_u * x_j, axis=1)
  lp = jnp.sum(jnp.minimum(d, 0.0) - jnp.log(1.0 + jnp.exp(-jnp.abs(d))))
  reg = 1e-4 * (jnp.sum(x_u ** 2) + jnp.sum(x_i ** 2) + jnp.sum(x_j ** 2))
  out[...] = jnp.reshape(reg - lp, (1, 1))


def _prep_params(P):
  f32 = jnp.float32
  I8 = jnp.eye(8, dtype=f32)
  bd = lambda w: jnp.kron(I8, w)
  tile = lambda v: jnp.tile(v, 8)[None]

  def split2(w):
    return w[:D], w[D:]

  pr = {}
  tabs = {}
  for tag, agg in (("u_", P["agg_u"]), ("v_", P["agg_v"])):
    w1a, w1b = split2(agg["w_r1_w"])
    tabs[tag] = P["r2e"] @ w1b
    pr[tag + "T40"] = bd(tabs[tag])
    att = agg["att"]
    a1a, a1b_w = split2(att["a1w"])
    pr[tag + "W1a"] = bd(w1a)
    pr[tag + "b1"] = tile(agg["w_r1_b"])
    pr[tag + "W2"] = bd(agg["w_r2_w"])
    pr[tag + "b2"] = tile(agg["w_r2_b"])
    pr[tag + "A1a"] = bd(a1a)
    pr[tag + "A1b"] = a1b_w
    pr[tag + "a1b"] = att["a1b"][None]
    pr[tag + "A2"] = bd(att["a2w"])
    pr[tag + "a2b"] = tile(att["a2b"])
    pr[tag + "a3"] = bd(att["a3w"])
  s1a, s1b_w = split2(P["soc_att"]["a1w"])
  pr["S1a"], pr["S1b"] = bd(s1a), s1b_w
  pr["s1b"] = P["soc_att"]["a1b"][None]
  pr["S2"], pr["s2b"] = bd(P["soc_att"]["a2w"]), tile(P["soc_att"]["a2b"])
  pr["s3"] = bd(P["soc_att"]["a3w"])
  for nm, key in (("Euh", "enc_uh"), ("Evh", "enc_vh"), ("Eu", "enc_u")):
    wa, wb = split2(P[key + "_w"])
    pr[nm + "A"], pr[nm + "B"] = wa, wb
  pr["euhb"] = P["enc_uh_b"][None]
  pr["evhb"] = P["enc_vh_b"][None]
  pr["eub"] = P["enc_u_b"][None]
  pr["Wur1"], pr["bur1"] = P["w_ur1_w"], P["w_ur1_b"][None]
  pr["Wvr1"], pr["bvr1"] = P["w_vr1_w"], P["w_vr1_b"][None]
  pr["E16"] = jnp.kron(I8, jnp.ones((1, D), f32))
  pr["E5"] = jnp.kron(I8, jnp.ones((1, NRAT), f32))
  pr["K40"] = (jnp.arange(8 * NRAT) % NRAT).astype(f32)[None]
  pr["F"] = jnp.kron(jnp.ones((8, 1), f32), jnp.eye(D))
  pr["G16"] = jnp.kron(jnp.ones((1, 8), f32), jnp.eye(D))
  nidx = jnp.arange(SLH).reshape(NPH, 8)
  pr["maskh"] = jnp.where(nidx < LH, 0.0, -1e30).astype(f32)
  sidx = jnp.arange(SLS).reshape(NPS, 8)
  pr["masks"] = jnp.where(sidx < LS, 0.0, -1e30).astype(f32)
  p2 = dict(Wur2=P["w_ur2_w"], bur2=P["w_ur2_b"][None],
            Wvr2=P["w_vr2_w"], bvr2=P["w_vr2_b"][None],
            g1=P["bn1_g"][None], b1=P["bn1_b"][None],
            g2=P["bn2_g"][None], b2=P["bn2_b"][None])
  return pr, p2, tabs["u_"], tabs["v_"]


def _full(x):
  return pl.BlockSpec(x.shape, lambda *_: (0,) * x.ndim)


def _tc_stage1(ehp_u, rsl_u, rep_u, socp, ehp_p, rsl_p, rep_p,
               ehp_n, rsl_n, rep_n, pr):
  eh_spec = pl.BlockSpec((C * NPH, 128), lambda i: (i, 0))
  rsl_spec = pl.BlockSpec((C * NPH, 8), lambda i: (i, 0))
  rep_spec = pl.BlockSpec((C, D), lambda i: (i, 0))
  soc_spec = pl.BlockSpec((C * NPS, 128), lambda i: (i, 0))
  f32 = jnp.float32
  nb = rep_u.shape[0]
  return pl.pallas_call(
      _tc1_body,
      grid=(nb // C,),
      in_specs=[eh_spec, rsl_spec, rep_spec, soc_spec,
                eh_spec, rsl_spec, rep_spec,
                eh_spec, rsl_spec, rep_spec, jax.tree.map(_full, pr)],
      out_specs=[rep_spec, rep_spec, rep_spec],
      out_shape=[jax.ShapeDtypeStruct((nb, D), f32)] * 3,
  )(ehp_u, rsl_u, rep_u, socp, ehp_p, rsl_p, rep_p,
    ehp_n, rsl_n, rep_n, pr)


def _tc_stage2(xu, xi, xj, p2):
  x_spec = pl.BlockSpec((B, D), lambda: (0, 0))
  return pl.pallas_call(
      _tc2_body,
      in_specs=[x_spec, x_spec, x_spec, jax.tree.map(_full, p2)],
      out_specs=pl.BlockSpec((1, 1), lambda: (0, 0)),
      out_shape=jax.ShapeDtypeStruct((1, 1), jnp.float32),
  )(xu, xi, xj, p2)


def kernel(nodes_u, nodes_pos, nodes_neg, hist_u, hist_ur, hist_v, hist_vr,
           soc_adj, params):
  pr, p2, _, _ = _prep_params(params)
  fl = lambda a: a.reshape(-1, 128)
  r8 = lambda a: a.reshape(-1, 8)
  h = B // 4
  xs = []
  for lo in (0, h, 2 * h, 3 * h):
    (eh_u, eh_p, eh_n, rs_u, rs_p, rs_n, soce, rp_u, rp_p,
     rp_n) = _sc_gather(
        lax.dynamic_slice_in_dim(nodes_u, lo, h),
        lax.dynamic_slice_in_dim(nodes_pos, lo, h),
        lax.dynamic_slice_in_dim(nodes_neg, lo, h),
        hist_u, hist_ur, hist_v, hist_vr, soc_adj,
        params["u2e"], params["v2e"])
    xs.append(_tc_stage1(fl(eh_u), r8(rs_u), rp_u, fl(soce),
                         fl(eh_p), r8(rs_p), rp_p,
                         fl(eh_n), r8(rs_n), rp_n, pr))
  xu, xi, xj = (jnp.concatenate(parts) for parts in zip(*xs))
  return _tc_stage2(xu, xi, xj, p2)[0, 0]
